# baseline (speedup 1.0000x reference)
import functools

import jax
import jax.numpy as jnp
from jax import lax
from jax.experimental import pallas as pl
from jax.experimental.pallas import tpu as pltpu
from jax.experimental.pallas import tpu_sc as plsc

_NC = 2
_NS = 16
_NW = _NC * _NS
_L = 16
_FCHUNK = 4096
_NACC = 4


@jax.jit
def _partials(ft, labels, ct):
    D, B = ft.shape
    V = ct.shape[1]
    dims_per_w = D // _NW
    nchunk = B // _FCHUNK
    ntask = dims_per_w * nchunk
    mesh = plsc.VectorSubcoreMesh(core_axis_name="c", subcore_axis_name="s")

    @functools.partial(
        pl.kernel,
        out_type=jax.ShapeDtypeStruct((_NW, _L), jnp.float32),
        mesh=mesh,
        scratch_types=[
            pltpu.VMEM((B,), jnp.int32),
            pltpu.VMEM((2, _FCHUNK), jnp.float32),
            pltpu.VMEM((V,), jnp.float32),
            pltpu.VMEM((_L,), jnp.float32),
            pltpu.SemaphoreType.DMA,
            pltpu.SemaphoreType.DMA,
        ],
        compiler_params=pltpu.CompilerParams(needs_layout_passes=False),
    )
    def sc_kernel(ft_hbm, labels_hbm, ct_hbm, out_hbm,
                  lab_v, frow_v, crow_v, acc_v, lab_sem, fsem):
        wid = lax.axis_index("s") * _NC + lax.axis_index("c")
        d0 = wid * dims_per_w

        lab_cp = pltpu.async_copy(labels_hbm, lab_v, lab_sem)
        pltpu.async_copy(
            ft_hbm.at[d0, pl.ds(0, _FCHUNK)], frow_v.at[0], fsem)
        lab_cp.wait()

        def one_task(t, buf, accs):
            d = d0 + t // nchunk
            c = t % nchunk

            @pl.when(c == 0)
            def _():
                pltpu.sync_copy(ct_hbm.at[d], crow_v)

            @pl.when(t + 1 < ntask)
            def _():
                tn = t + 1
                pltpu.async_copy(
                    ft_hbm.at[d0 + tn // nchunk,
                              pl.ds((tn % nchunk) * _FCHUNK, _FCHUNK)],
                    frow_v.at[1 - buf], fsem)

            pltpu.make_async_copy(
                ft_hbm.at[d0, pl.ds(0, _FCHUNK)], frow_v.at[buf], fsem
            ).wait()

            base = c * _FCHUNK

            @plsc.parallel_loop(0, _FCHUNK, step=_NACC * _L, unroll=2,
                                carry=accs)
            def accs_out(i, acc_in):
                out = []
                for k in range(_NACC):
                    off = base + i + k * _L
                    idx = lab_v[pl.ds(off, _L)]
                    g = plsc.load_gather(crow_v, [idx])
                    f = frow_v[buf, pl.ds(i + k * _L, _L)]
                    dd = f - g
                    out.append(acc_in[k] + dd * dd)
                return tuple(out)

            return accs_out

        def pair_body(k, accs):
            accs = one_task(2 * k, 0, accs)
            accs = one_task(2 * k + 1, 1, accs)
            return accs

        zeros = tuple(jnp.zeros((_L,), jnp.float32) for _ in range(_NACC))
        accs = lax.fori_loop(0, ntask // 2, pair_body, zeros)

        total = accs[0]
        for k in range(1, _NACC):
            total = total + accs[k]
        acc_v[...] = total
        pltpu.sync_copy(acc_v, out_hbm.at[wid])

    return sc_kernel(ft, labels, ct)


def kernel(features, labels, centers):
    B = features.shape[0]
    partials = _partials(features.T, labels.astype(jnp.int32), centers.T)
    return jnp.sum(partials) / 2.0 / B
